# baseline (speedup 1.0000x reference)
import functools

import jax
import jax.numpy as jnp
from jax import lax
from jax.experimental import pallas as pl
from jax.experimental.pallas import tpu as pltpu
from jax.experimental.pallas import tpu_sc as plsc

NC = 2
NS = 16
NW = NC * NS
K = 80
NB = 5
GLAT = 4


def _mesh():
    return plsc.VectorSubcoreMesh(core_axis_name="c", subcore_axis_name="s")


@functools.lru_cache(maxsize=None)
def _deg_kernel(E, NPAD):
    EPW = E // NW
    NCH = EPW // K
    SPT = NPAD // NS

    @functools.partial(
        pl.kernel,
        out_type=jax.ShapeDtypeStruct((NC * NPAD,), jnp.float32),
        mesh=_mesh(),
        scratch_types=[
            pltpu.VMEM_SHARED((NPAD,), jnp.float32),
            pltpu.VMEM((SPT,), jnp.float32),
            pltpu.VMEM((K,), jnp.float32),
            [pltpu.VMEM((K,), jnp.int32) for _ in range(NB)],
            [pltpu.SemaphoreType.DMA for _ in range(NB)],
        ],
    )
    def deg_kernel(dstr_hbm, out_hbm, deg_sh, zbuf, ones_v, dbufs, isem):
        c = lax.axis_index("c")
        s = lax.axis_index("s")
        wid = c * NS + s

        def zfill(i, _):
            zbuf[pl.ds(i * 16, 16)] = jnp.zeros((16,), jnp.float32)
            return 0

        lax.fori_loop(0, SPT // 16, zfill, 0)

        def ofill(i, _):
            ones_v[pl.ds(i * 16, 16)] = jnp.ones((16,), jnp.float32)
            return 0

        lax.fori_loop(0, K // 16, ofill, 0)

        pltpu.sync_copy(zbuf, deg_sh.at[pl.ds(s * SPT, SPT)])
        plsc.subcore_barrier()

        def iissue(chunk, b):
            pltpu.async_copy(dstr_hbm.at[wid, chunk], dbufs[b], isem[b])

        def iwait(b):
            pltpu.make_async_copy(dstr_hbm.at[0, 0], dbufs[b],
                                  isem[b]).wait()

        for p in range(GLAT):
            iissue(p, p)

        def lap(j, _):
            for b in range(NB):
                t = j * NB + b
                bp = (b + GLAT) % NB

                @pl.when(t + GLAT < NCH)
                def _():
                    iissue(t + GLAT, bp)

                iwait(b)
                pltpu.sync_copy(ones_v, deg_sh.at[dbufs[b]], add=True)
            return 0

        lax.fori_loop(0, NCH // NB, lap, 0)
        plsc.subcore_barrier()

        pltpu.sync_copy(deg_sh.at[pl.ds(s * SPT, SPT)], zbuf)
        pltpu.sync_copy(zbuf, out_hbm.at[pl.ds(c * NPAD + s * SPT, SPT)])

    return deg_kernel


@functools.lru_cache(maxsize=None)
def _agg_kernel(E, N, H):
    EPW = E // NW
    NCH = EPW // K
    RPT = (N // NS) & ~7
    TAIL = N - NS * RPT

    @functools.partial(
        pl.kernel,
        out_type=jax.ShapeDtypeStruct((NC, N, H), jnp.float32),
        mesh=_mesh(),
        compiler_params=pltpu.CompilerParams(use_tc_tiling_on_sc=False),
        scratch_types=[
            pltpu.VMEM_SHARED((N, H), jnp.float32),
            pltpu.VMEM((RPT, H), jnp.float32),
            [pltpu.VMEM((K, H), jnp.float32) for _ in range(NB)],
            [pltpu.VMEM((K,), jnp.int32) for _ in range(NB)],
            [pltpu.VMEM((K,), jnp.int32) for _ in range(NB)],
            [pltpu.SemaphoreType.DMA for _ in range(NB)],
            [pltpu.SemaphoreType.DMA for _ in range(NB)],
            [pltpu.SemaphoreType.DMA for _ in range(NB)],
        ],
    )
    def agg_kernel(hs_hbm, srcr_hbm, dstr_hbm, out_hbm, acc_sh, stage_v,
                   rows, sbufs, dbufs, gsem, isem, jsem):
        c = lax.axis_index("c")
        s = lax.axis_index("s")
        wid = c * NS + s

        pltpu.sync_copy(hs_hbm.at[pl.ds(s * RPT, RPT)], stage_v)
        pltpu.sync_copy(stage_v, acc_sh.at[pl.ds(s * RPT, RPT)])
        if TAIL:
            @pl.when(s == NS - 1)
            def _():
                pltpu.sync_copy(hs_hbm.at[pl.ds(NS * RPT, TAIL)],
                                stage_v.at[pl.ds(0, TAIL)])
                pltpu.sync_copy(stage_v.at[pl.ds(0, TAIL)],
                                acc_sh.at[pl.ds(NS * RPT, TAIL)])
        plsc.subcore_barrier()

        def iissue(chunk, b):
            pltpu.async_copy(dstr_hbm.at[wid, chunk], dbufs[b], isem[b])

        def iwait(b):
            pltpu.make_async_copy(dstr_hbm.at[0, 0], dbufs[b],
                                  isem[b]).wait()

        def jissue(chunk, b):
            pltpu.async_copy(srcr_hbm.at[wid, chunk], sbufs[b], jsem[b])

        def jwait(b):
            pltpu.make_async_copy(srcr_hbm.at[0, 0], sbufs[b],
                                  jsem[b]).wait()

        def gissue(b):
            pltpu.async_copy(hs_hbm.at[sbufs[b]], rows[b], gsem[b])

        def gwait(b):
            pltpu.make_async_copy(hs_hbm.at[sbufs[b]], rows[b],
                                  gsem[b]).wait()

        for p in range(NB):
            jissue(p, p)
        for p in range(GLAT):
            iissue(p, p)
            jwait(p)
            gissue(p)

        def lap(j, _):
            for b in range(NB):
                t = j * NB + b
                bp = (b + GLAT) % NB

                @pl.when(t + GLAT < NCH)
                def _():
                    iissue(t + GLAT, bp)
                    jwait(bp)
                    gissue(bp)

                gwait(b)
                iwait(b)
                pltpu.sync_copy(rows[b], acc_sh.at[dbufs[b]], add=True)

                @pl.when(t + NB < NCH)
                def _():
                    jissue(t + NB, b)
            return 0

        lax.fori_loop(0, NCH // NB, lap, 0)
        plsc.subcore_barrier()

        pltpu.sync_copy(acc_sh.at[pl.ds(s * RPT, RPT)], stage_v)
        pltpu.sync_copy(stage_v, out_hbm.at[c, pl.ds(s * RPT, RPT)])
        if TAIL:
            @pl.when(s == NS - 1)
            def _():
                pltpu.sync_copy(acc_sh.at[pl.ds(NS * RPT, TAIL)],
                                stage_v.at[pl.ds(0, TAIL)])
                pltpu.sync_copy(stage_v.at[pl.ds(0, TAIL)],
                                out_hbm.at[c, pl.ds(NS * RPT, TAIL)])

    return agg_kernel


def _tc_b(deg_ref, x_ref, w_ref, hs_ref, dinv_ref):
    n = x_ref.shape[0]
    deg = deg_ref[0] + deg_ref[1] + 1.0
    dinv = lax.rsqrt(deg)[:n]
    h = jnp.dot(x_ref[...], w_ref[...], preferred_element_type=jnp.float32)
    hs_ref[...] = h * dinv
    dinv_ref[...] = dinv


def _tc_d(acc_ref, hs_ref, dinv_ref, b_ref, w_ref, x1_ref, hs2_ref):
    agg = acc_ref[0] + acc_ref[1] - hs_ref[...]
    x1 = jnp.maximum(dinv_ref[...] * agg + b_ref[...], 0.0)
    x1_ref[...] = x1
    hs2_ref[...] = jnp.dot(
        x1, w_ref[...], preferred_element_type=jnp.float32) * dinv_ref[...]


def _tc_f(acc_ref, hs_ref, dinv_ref, b_ref, x1_ref, wl_ref, bl_ref, out_ref):
    agg = acc_ref[0] + acc_ref[1] - hs_ref[...]
    x2 = jnp.maximum(dinv_ref[...] * agg + b_ref[...], 0.0)
    xc = jnp.concatenate([x1_ref[...], x2], axis=1)
    out_ref[...] = jnp.maximum(
        jnp.dot(xc, wl_ref[...], preferred_element_type=jnp.float32)
        + bl_ref[...], 0.0)


def kernel(x, edge_index, W1, b1, W2, b2, Wl, bl):
    N, D = x.shape
    H = W1.shape[1]
    C = Wl.shape[1]
    E = edge_index.shape[1]
    NPAD = ((N + 8 * NS - 1) // (8 * NS)) * (8 * NS)
    EPW = E // NW
    NCH = EPW // K

    srcr = edge_index[0].reshape(NW, NCH, K)
    dstr = edge_index[1].reshape(NW, NCH, K)

    deg = _deg_kernel(E, NPAD)(dstr)
    deg3 = deg.reshape(NC, NPAD, 1)

    hs1, dinv = pl.pallas_call(
        _tc_b,
        out_shape=[
            jax.ShapeDtypeStruct((N, H), jnp.float32),
            jax.ShapeDtypeStruct((N, 1), jnp.float32),
        ],
    )(deg3, x, W1)

    agg = _agg_kernel(E, N, H)
    acc1 = agg(hs1, srcr, dstr)

    x1, hs2 = pl.pallas_call(
        _tc_d,
        out_shape=[
            jax.ShapeDtypeStruct((N, H), jnp.float32),
            jax.ShapeDtypeStruct((N, H), jnp.float32),
        ],
    )(acc1, hs1, dinv, b1.reshape(1, H), W2)

    acc2 = agg(hs2, srcr, dstr)

    out = pl.pallas_call(
        _tc_f,
        out_shape=jax.ShapeDtypeStruct((N, C), jnp.float32),
    )(acc2, hs2, dinv, b2.reshape(1, H), x1, Wl, bl.reshape(1, C))
    return out


# speedup vs baseline: 1.2998x; 1.2998x over previous
import functools

import jax
import jax.numpy as jnp
from jax import lax
from jax.experimental import pallas as pl
from jax.experimental.pallas import tpu as pltpu
from jax.experimental.pallas import tpu_sc as plsc

NC = 2
NS = 16
NW = NC * NS
K = 80
NB = 5
GLAT = 2
GLAT_DEG = 4


def _mesh():
    return plsc.VectorSubcoreMesh(core_axis_name="c", subcore_axis_name="s")


@functools.lru_cache(maxsize=None)
def _deg_kernel(E, NPAD):
    EPW = E // NW
    NCH = EPW // K
    SPT = NPAD // NS

    @functools.partial(
        pl.kernel,
        out_type=jax.ShapeDtypeStruct((NC * NPAD,), jnp.float32),
        mesh=_mesh(),
        scratch_types=[
            pltpu.VMEM_SHARED((NPAD,), jnp.float32),
            pltpu.VMEM((SPT,), jnp.float32),
            pltpu.VMEM((K,), jnp.float32),
            [pltpu.VMEM((K,), jnp.int32) for _ in range(NB)],
            [pltpu.SemaphoreType.DMA for _ in range(NB)],
        ],
    )
    def deg_kernel(dstr_hbm, out_hbm, deg_sh, zbuf, ones_v, dbufs, isem):
        c = lax.axis_index("c")
        s = lax.axis_index("s")
        wid = c * NS + s

        def zfill(i, _):
            zbuf[pl.ds(i * 16, 16)] = jnp.zeros((16,), jnp.float32)
            return 0

        lax.fori_loop(0, SPT // 16, zfill, 0)

        def ofill(i, _):
            ones_v[pl.ds(i * 16, 16)] = jnp.ones((16,), jnp.float32)
            return 0

        lax.fori_loop(0, K // 16, ofill, 0)

        pltpu.sync_copy(zbuf, deg_sh.at[pl.ds(s * SPT, SPT)])
        plsc.subcore_barrier()

        def iissue(chunk, b):
            pltpu.async_copy(dstr_hbm.at[wid, chunk], dbufs[b], isem[b])

        def iwait(b):
            pltpu.make_async_copy(dstr_hbm.at[0, 0], dbufs[b],
                                  isem[b]).wait()

        for p in range(GLAT_DEG):
            iissue(p, p)

        def lap(j, _):
            for b in range(NB):
                t = j * NB + b
                bp = (b + GLAT_DEG) % NB

                @pl.when(t + GLAT_DEG < NCH)
                def _():
                    iissue(t + GLAT_DEG, bp)

                iwait(b)
                pltpu.sync_copy(ones_v, deg_sh.at[dbufs[b]], add=True)
            return 0

        lax.fori_loop(0, NCH // NB, lap, 0)
        plsc.subcore_barrier()

        pltpu.sync_copy(deg_sh.at[pl.ds(s * SPT, SPT)], zbuf)
        pltpu.sync_copy(zbuf, out_hbm.at[pl.ds(c * NPAD + s * SPT, SPT)])

    return deg_kernel


@functools.lru_cache(maxsize=None)
def _agg_kernel(E, N, H):
    EPW = E // NW
    NCH = EPW // K
    RPT = (N // NS) & ~7
    TAIL = N - NS * RPT

    @functools.partial(
        pl.kernel,
        out_type=jax.ShapeDtypeStruct((NC, N, H), jnp.float32),
        mesh=_mesh(),
        compiler_params=pltpu.CompilerParams(use_tc_tiling_on_sc=False),
        scratch_types=[
            pltpu.VMEM_SHARED((N, H), jnp.float32),
            pltpu.VMEM((RPT, H), jnp.float32),
            [pltpu.VMEM((K, H), jnp.float32) for _ in range(NB)],
            [pltpu.VMEM((K,), jnp.int32) for _ in range(NB)],
            [pltpu.VMEM((K,), jnp.int32) for _ in range(NB)],
            [pltpu.SemaphoreType.DMA for _ in range(NB)],
            [pltpu.SemaphoreType.DMA for _ in range(NB)],
            [pltpu.SemaphoreType.DMA for _ in range(NB)],
        ],
    )
    def agg_kernel(hs_hbm, srcr_hbm, dstr_hbm, out_hbm, acc_sh, stage_v,
                   rows, sbufs, dbufs, gsem, isem, jsem):
        c = lax.axis_index("c")
        s = lax.axis_index("s")
        wid = c * NS + s

        pltpu.sync_copy(hs_hbm.at[pl.ds(s * RPT, RPT)], stage_v)
        pltpu.sync_copy(stage_v, acc_sh.at[pl.ds(s * RPT, RPT)])
        if TAIL:
            @pl.when(s == NS - 1)
            def _():
                pltpu.sync_copy(hs_hbm.at[pl.ds(NS * RPT, TAIL)],
                                stage_v.at[pl.ds(0, TAIL)])
                pltpu.sync_copy(stage_v.at[pl.ds(0, TAIL)],
                                acc_sh.at[pl.ds(NS * RPT, TAIL)])
        plsc.subcore_barrier()

        def iissue(chunk, b):
            pltpu.async_copy(dstr_hbm.at[wid, chunk], dbufs[b], isem[b])

        def iwait(b):
            pltpu.make_async_copy(dstr_hbm.at[0, 0], dbufs[b],
                                  isem[b]).wait()

        def jissue(chunk, b):
            pltpu.async_copy(srcr_hbm.at[wid, chunk], sbufs[b], jsem[b])

        def jwait(b):
            pltpu.make_async_copy(srcr_hbm.at[0, 0], sbufs[b],
                                  jsem[b]).wait()

        def gissue(b):
            pltpu.async_copy(hs_hbm.at[sbufs[b]], rows[b], gsem[b])

        def gwait(b):
            pltpu.make_async_copy(hs_hbm.at[sbufs[b]], rows[b],
                                  gsem[b]).wait()

        for p in range(NB):
            jissue(p, p)
        for p in range(GLAT):
            iissue(p, p)
            jwait(p)
            gissue(p)

        def lap(j, _):
            for b in range(NB):
                t = j * NB + b
                bp = (b + GLAT) % NB

                @pl.when(t + GLAT < NCH)
                def _():
                    iissue(t + GLAT, bp)
                    jwait(bp)
                    gissue(bp)

                gwait(b)
                iwait(b)
                pltpu.sync_copy(rows[b], acc_sh.at[dbufs[b]], add=True)

                @pl.when(t + NB < NCH)
                def _():
                    jissue(t + NB, b)
            return 0

        lax.fori_loop(0, NCH // NB, lap, 0)
        plsc.subcore_barrier()

        pltpu.sync_copy(acc_sh.at[pl.ds(s * RPT, RPT)], stage_v)
        pltpu.sync_copy(stage_v, out_hbm.at[c, pl.ds(s * RPT, RPT)])
        if TAIL:
            @pl.when(s == NS - 1)
            def _():
                pltpu.sync_copy(acc_sh.at[pl.ds(NS * RPT, TAIL)],
                                stage_v.at[pl.ds(0, TAIL)])
                pltpu.sync_copy(stage_v.at[pl.ds(0, TAIL)],
                                out_hbm.at[c, pl.ds(NS * RPT, TAIL)])

    return agg_kernel


def _tc_b(deg_ref, x_ref, w_ref, hs_ref, dinv_ref):
    n = x_ref.shape[0]
    deg = deg_ref[0] + deg_ref[1] + 1.0
    dinv = lax.rsqrt(deg)[:n]
    h = jnp.dot(x_ref[...], w_ref[...], preferred_element_type=jnp.float32)
    hs_ref[...] = h * dinv
    dinv_ref[...] = dinv


def _tc_d(acc_ref, hs_ref, dinv_ref, b_ref, w_ref, x1_ref, hs2_ref):
    agg = acc_ref[0] + acc_ref[1] - hs_ref[...]
    x1 = jnp.maximum(dinv_ref[...] * agg + b_ref[...], 0.0)
    x1_ref[...] = x1
    hs2_ref[...] = jnp.dot(
        x1, w_ref[...], preferred_element_type=jnp.float32) * dinv_ref[...]


def _tc_f(acc_ref, hs_ref, dinv_ref, b_ref, x1_ref, wl_ref, bl_ref, out_ref):
    agg = acc_ref[0] + acc_ref[1] - hs_ref[...]
    x2 = jnp.maximum(dinv_ref[...] * agg + b_ref[...], 0.0)
    xc = jnp.concatenate([x1_ref[...], x2], axis=1)
    out_ref[...] = jnp.maximum(
        jnp.dot(xc, wl_ref[...], preferred_element_type=jnp.float32)
        + bl_ref[...], 0.0)


def kernel(x, edge_index, W1, b1, W2, b2, Wl, bl):
    N, D = x.shape
    H = W1.shape[1]
    C = Wl.shape[1]
    E = edge_index.shape[1]
    NPAD = ((N + 8 * NS - 1) // (8 * NS)) * (8 * NS)
    EPW = E // NW
    NCH = EPW // K

    srcr = edge_index[0].reshape(NW, NCH, K)
    dstr = edge_index[1].reshape(NW, NCH, K)

    deg = _deg_kernel(E, NPAD)(dstr)
    deg3 = deg.reshape(NC, NPAD, 1)

    hs1, dinv = pl.pallas_call(
        _tc_b,
        out_shape=[
            jax.ShapeDtypeStruct((N, H), jnp.float32),
            jax.ShapeDtypeStruct((N, 1), jnp.float32),
        ],
    )(deg3, x, W1)

    agg = _agg_kernel(E, N, H)
    acc1 = agg(hs1, srcr, dstr)

    x1, hs2 = pl.pallas_call(
        _tc_d,
        out_shape=[
            jax.ShapeDtypeStruct((N, H), jnp.float32),
            jax.ShapeDtypeStruct((N, H), jnp.float32),
        ],
    )(acc1, hs1, dinv, b1.reshape(1, H), W2)

    acc2 = agg(hs2, srcr, dstr)

    out = pl.pallas_call(
        _tc_f,
        out_shape=jax.ShapeDtypeStruct((N, C), jnp.float32),
    )(acc2, hs2, dinv, b2.reshape(1, H), x1, Wl, bl.reshape(1, C))
    return out
